# Initial kernel scaffold; baseline (speedup 1.0000x reference)
import jax
import jax.numpy as jnp
from jax.experimental import pallas as pl


def kernel(embeddings, labels):
    raise NotImplementedError("write your pallas kernel here")


# speedup vs baseline: 1123.1550x; 1123.1550x over previous
import jax
import jax.numpy as jnp
from jax.experimental import pallas as pl

_MARGIN = 1.0
_B = 1024
_NPAIRS = _B * (_B - 1) // 2
_CONTRACT_LAST = (((1,), (1,)), ((), ()))


def _loss_kernel(emb_ref, lab_ref, out_ref):
    e = emb_ref[:]
    lab = lab_ref[:]

    m = jnp.max(lab, axis=1, keepdims=True)
    col = jax.lax.broadcasted_iota(jnp.int32, lab.shape, 1)
    idx = jnp.min(jnp.where(lab == m, col, lab.shape[1]), axis=1, keepdims=True)
    onehot = (col == idx).astype(jnp.float32)

    g = jax.lax.dot_general(e, e, _CONTRACT_LAST,
                            preferred_element_type=jnp.float32)
    sq = e * e
    n_col = jnp.sum(sq, axis=1, keepdims=True)
    ones = jnp.ones((1, e.shape[1]), jnp.float32)
    n_row = jax.lax.dot_general(ones, sq, _CONTRACT_LAST,
                                preferred_element_type=jnp.float32)
    d2 = jnp.maximum(n_col + n_row - 2.0 * g, 0.0)

    eqf = jax.lax.dot_general(onehot, onehot, _CONTRACT_LAST,
                              preferred_element_type=jnp.float32)

    neg = jnp.square(jnp.maximum(_MARGIN - jnp.sqrt(d2), 0.0))
    loss = eqf * d2 + (1.0 - eqf) * neg
    row_sums = jnp.sum(loss, axis=1, keepdims=True)
    total = jnp.sum(row_sums, axis=0, keepdims=True)
    out_ref[:, :] = total / (2.0 * _NPAIRS)


def kernel(embeddings, labels):
    out = pl.pallas_call(
        _loss_kernel,
        out_shape=jax.ShapeDtypeStruct((1, 1), jnp.float32),
    )(embeddings, labels)
    return out[0, 0]


# speedup vs baseline: 1167.6391x; 1.0396x over previous
import jax
import jax.numpy as jnp
from jax.experimental import pallas as pl

_MARGIN = 1.0
_B = 1024
_NPAIRS = _B * (_B - 1) // 2
_CONTRACT_LAST = (((1,), (1,)), ((), ()))


def _loss_kernel(emb_ref, lab_ref, out_ref):
    e = emb_ref[:]
    lab = lab_ref[:]

    m = jnp.max(lab, axis=1, keepdims=True)
    col = jax.lax.broadcasted_iota(jnp.int32, lab.shape, 1)
    idx = jnp.min(jnp.where(lab == m, col, lab.shape[1]), axis=1, keepdims=True)
    idx_f = idx.astype(jnp.float32)

    g = jax.lax.dot_general(e, e, _CONTRACT_LAST,
                            preferred_element_type=jnp.float32)
    sq = e * e
    n_col = jnp.sum(sq, axis=1, keepdims=True)
    ones = jnp.ones((1, e.shape[1]), jnp.float32)
    n_row = jax.lax.dot_general(ones, sq, _CONTRACT_LAST,
                                preferred_element_type=jnp.float32)
    d2 = jnp.maximum(n_col + n_row - 2.0 * g, 0.0)

    one = jnp.ones((1, 1), jnp.float32)
    idx_row = jax.lax.dot_general(one, idx_f, _CONTRACT_LAST,
                                  preferred_element_type=jnp.float32)
    eq = idx_f == idx_row

    neg = jnp.square(jnp.maximum(_MARGIN - jnp.sqrt(d2), 0.0))
    loss = jnp.where(eq, d2, neg)
    row_sums = jnp.sum(loss, axis=1, keepdims=True)
    total = jnp.sum(row_sums, axis=0, keepdims=True)
    out_ref[:, :] = total / (2.0 * _NPAIRS)


def kernel(embeddings, labels):
    out = pl.pallas_call(
        _loss_kernel,
        out_shape=jax.ShapeDtypeStruct((1, 1), jnp.float32),
    )(embeddings, labels)
    return out[0, 0]


# speedup vs baseline: 1290.4312x; 1.1052x over previous
import jax
import jax.numpy as jnp
from jax.experimental import pallas as pl

_MARGIN = 1.0
_B = 1024
_NPAIRS = _B * (_B - 1) // 2
_CONTRACT_LAST = (((1,), (1,)), ((), ()))


def _loss_kernel(emb_ref, lab_ref, out_ref):
    e = emb_ref[:]
    lab = lab_ref[:]

    m = jnp.max(lab, axis=1, keepdims=True)
    col = jax.lax.broadcasted_iota(jnp.int32, lab.shape, 1)
    idx = jnp.min(jnp.where(lab == m, col, lab.shape[1]), axis=1, keepdims=True)
    idx_f = idx.astype(jnp.float32)

    g = jax.lax.dot_general(e, e, _CONTRACT_LAST,
                            preferred_element_type=jnp.float32)
    sq = e * e
    n_col = jnp.sum(sq, axis=1, keepdims=True)
    ones = jnp.ones((1, e.shape[1]), jnp.float32)
    n_row = jax.lax.dot_general(ones, sq, _CONTRACT_LAST,
                                preferred_element_type=jnp.float32)
    d2 = jnp.maximum(n_col + n_row - 2.0 * g, 0.0)

    one = jnp.ones((1, 1), jnp.float32)
    idx_row = jax.lax.dot_general(one, idx_f, _CONTRACT_LAST,
                                  preferred_element_type=jnp.float32)
    eq = idx_f == idx_row

    s = d2 * jax.lax.rsqrt(d2 + 1e-12)
    t = jnp.maximum(_MARGIN - s, 0.0)
    loss = jnp.where(eq, d2, t * t)
    row_sums = jnp.sum(loss, axis=1, keepdims=True)
    total = jnp.sum(row_sums, axis=0, keepdims=True)
    out_ref[:, :] = total / (2.0 * _NPAIRS)


def kernel(embeddings, labels):
    out = pl.pallas_call(
        _loss_kernel,
        out_shape=jax.ShapeDtypeStruct((1, 1), jnp.float32),
    )(embeddings, labels)
    return out[0, 0]


# speedup vs baseline: 1295.6285x; 1.0040x over previous
import jax
import jax.numpy as jnp
from jax.experimental import pallas as pl

_MARGIN = 1.0
_B = 1024
_NPAIRS = _B * (_B - 1) // 2
_CONTRACT_LAST = (((1,), (1,)), ((), ()))


def _loss_kernel(emb_ref, lab_ref, out_ref):
    e = emb_ref[:]
    lab = lab_ref[:]

    m = jnp.max(lab, axis=1, keepdims=True)
    col = jax.lax.broadcasted_iota(jnp.int32, lab.shape, 1)
    idx = jnp.min(jnp.where(lab == m, col, lab.shape[1]), axis=1, keepdims=True)
    idx_f = idx.astype(jnp.float32)

    n_vec = jnp.sum(e * e, axis=1, keepdims=True)
    ones_col = jnp.ones((_B, 1), jnp.float32)
    a_aug = jnp.concatenate([-2.0 * e, n_vec, ones_col], axis=1)
    b_aug = jnp.concatenate([e, ones_col, n_vec], axis=1)
    d2 = jnp.maximum(
        jax.lax.dot_general(a_aug, b_aug, _CONTRACT_LAST,
                            preferred_element_type=jnp.float32), 0.0)

    one = jnp.ones((1, 1), jnp.float32)
    idx_row = jax.lax.dot_general(one, idx_f, _CONTRACT_LAST,
                                  preferred_element_type=jnp.float32)
    eq = idx_f == idx_row

    s = d2 * jax.lax.rsqrt(d2 + 1e-12)
    t = jnp.maximum(_MARGIN - s, 0.0)
    loss = jnp.where(eq, d2, t * t)

    row_sums = jax.lax.dot_general(loss, jnp.ones((1, _B), jnp.float32),
                                   _CONTRACT_LAST,
                                   preferred_element_type=jnp.float32)
    total = jnp.sum(row_sums, axis=0, keepdims=True)
    out_ref[:, :] = total / (2.0 * _NPAIRS)


def kernel(embeddings, labels):
    out = pl.pallas_call(
        _loss_kernel,
        out_shape=jax.ShapeDtypeStruct((1, 1), jnp.float32),
    )(embeddings, labels)
    return out[0, 0]


# speedup vs baseline: 1545.3997x; 1.1928x over previous
import jax
import jax.numpy as jnp
from jax.experimental import pallas as pl

_MARGIN = 1.0
_B = 1024
_T = 128
_NT = _B // _T
_NPAIRS = _B * (_B - 1) // 2
_CONTRACT_LAST = (((1,), (1,)), ((), ()))


def _loss_kernel(emb_ref, lab_ref, out_ref):
    e = emb_ref[:]
    lab = lab_ref[:]

    m = jnp.max(lab, axis=1, keepdims=True)
    col = jax.lax.broadcasted_iota(jnp.int32, lab.shape, 1)
    idx = jnp.min(jnp.where(lab == m, col, lab.shape[1]), axis=1, keepdims=True)
    idx_f = idx.astype(jnp.float32)

    one = jnp.ones((1, 1), jnp.float32)
    idx_row = jax.lax.dot_general(one, idx_f, _CONTRACT_LAST,
                                  preferred_element_type=jnp.float32)

    n_vec = jnp.sum(e * e, axis=1, keepdims=True)
    ones_col = jnp.ones((_B, 1), jnp.float32)
    a_aug = jnp.concatenate([-2.0 * e, n_vec, ones_col], axis=1)
    b_aug = jnp.concatenate([e, ones_col, n_vec], axis=1)

    acc_off = jnp.zeros((_T, _T), jnp.float32)
    acc_diag = jnp.zeros((_T, _T), jnp.float32)
    for bi in range(_NT):
        a_blk = a_aug[bi * _T:(bi + 1) * _T, :]
        idc = idx_f[bi * _T:(bi + 1) * _T, :]
        for bj in range(bi, _NT):
            b_blk = b_aug[bj * _T:(bj + 1) * _T, :]
            idr = idx_row[:, bj * _T:(bj + 1) * _T]
            d2 = jnp.maximum(
                jax.lax.dot_general(a_blk, b_blk, _CONTRACT_LAST,
                                    preferred_element_type=jnp.float32), 0.0)
            s = d2 * jax.lax.rsqrt(d2 + 1e-12)
            t = jnp.maximum(_MARGIN - s, 0.0)
            loss_t = jnp.where(idc == idr, d2, t * t)
            if bi == bj:
                acc_diag = acc_diag + loss_t
            else:
                acc_off = acc_off + loss_t
    tot = acc_off + 0.5 * acc_diag
    row_sums = jnp.sum(tot, axis=1, keepdims=True)
    total = jnp.sum(row_sums, axis=0, keepdims=True)
    out_ref[:, :] = total / _NPAIRS


def kernel(embeddings, labels):
    out = pl.pallas_call(
        _loss_kernel,
        out_shape=jax.ShapeDtypeStruct((1, 1), jnp.float32),
    )(embeddings, labels)
    return out[0, 0]


# speedup vs baseline: 1584.1841x; 1.0251x over previous
import jax
import jax.numpy as jnp
from jax.experimental import pallas as pl

_MARGIN = 1.0
_B = 1024
_T = 128
_NT = _B // _T
_NPAIRS = _B * (_B - 1) // 2
_CONTRACT_LAST = (((1,), (1,)), ((), ()))


def _loss_kernel(emb_ref, lab_ref, out_ref):
    e = emb_ref[:]
    lab = lab_ref[:]

    m = jnp.max(lab, axis=1, keepdims=True)
    col = jax.lax.broadcasted_iota(jnp.int32, lab.shape, 1)
    w = jax.lax.bitcast_convert_type((127 - col) << 23, jnp.float32)
    mw = jnp.where(lab == m, w, 0.0)
    z = jax.lax.dot_general(mw, jnp.ones((1, lab.shape[1]), jnp.float32),
                            _CONTRACT_LAST,
                            preferred_element_type=jnp.float32)
    zbits = jax.lax.bitcast_convert_type(z, jnp.int32)
    idx_f = (127 - (zbits >> 23)).astype(jnp.float32)

    one = jnp.ones((1, 1), jnp.float32)
    idx_row = jax.lax.dot_general(one, idx_f, _CONTRACT_LAST,
                                  preferred_element_type=jnp.float32)

    n_vec = jnp.sum(e * e, axis=1, keepdims=True)
    ones_col = jnp.ones((_B, 1), jnp.float32)
    a_aug = jnp.concatenate([-2.0 * e, n_vec, ones_col], axis=1)
    b_aug = jnp.concatenate([e, ones_col, n_vec], axis=1)

    acc_off = jnp.zeros((_T, _T), jnp.float32)
    acc_diag = jnp.zeros((_T, _T), jnp.float32)
    for bi in range(_NT):
        a_blk = a_aug[bi * _T:(bi + 1) * _T, :]
        idc = idx_f[bi * _T:(bi + 1) * _T, :]
        for bj in range(bi, _NT):
            b_blk = b_aug[bj * _T:(bj + 1) * _T, :]
            idr = idx_row[:, bj * _T:(bj + 1) * _T]
            d2 = jnp.maximum(
                jax.lax.dot_general(a_blk, b_blk, _CONTRACT_LAST,
                                    preferred_element_type=jnp.float32), 0.0)
            s = d2 * jax.lax.rsqrt(d2 + 1e-12)
            t = jnp.maximum(_MARGIN - s, 0.0)
            loss_t = jnp.where(idc == idr, d2, t * t)
            if bi == bj:
                acc_diag = acc_diag + loss_t
            else:
                acc_off = acc_off + loss_t
    tot = acc_off + 0.5 * acc_diag
    row_sums = jnp.sum(tot, axis=1, keepdims=True)
    total = jnp.sum(row_sums, axis=0, keepdims=True)
    out_ref[:, :] = total / _NPAIRS


def kernel(embeddings, labels):
    out = pl.pallas_call(
        _loss_kernel,
        out_shape=jax.ShapeDtypeStruct((1, 1), jnp.float32),
    )(embeddings, labels)
    return out[0, 0]


# speedup vs baseline: 1585.4844x; 1.0008x over previous
import jax
import jax.numpy as jnp
from jax.experimental import pallas as pl

_MARGIN = 1.0
_B = 1024
_T = 128
_NT = _B // _T
_NPAIRS = _B * (_B - 1) // 2
_CONTRACT_LAST = (((1,), (1,)), ((), ()))


def _loss_kernel(emb_ref, lab_ref, out_ref):
    e = emb_ref[:]
    lab = lab_ref[:]

    m = jnp.max(lab, axis=1, keepdims=True)
    col = jax.lax.broadcasted_iota(jnp.int32, (1, lab.shape[1]), 1)
    w = jax.lax.bitcast_convert_type((127 - col) << 23, jnp.float32)
    mw = jnp.where(lab == m, w, 0.0)
    z = jax.lax.dot_general(mw, jnp.ones((1, lab.shape[1]), jnp.float32),
                            _CONTRACT_LAST,
                            preferred_element_type=jnp.float32)
    zbits = jax.lax.bitcast_convert_type(z, jnp.int32)
    idx_f = (127 - (zbits >> 23)).astype(jnp.float32)

    one = jnp.ones((1, 1), jnp.float32)
    idx_row = jax.lax.dot_general(one, idx_f, _CONTRACT_LAST,
                                  preferred_element_type=jnp.float32)

    n_vec = jnp.sum(e * e, axis=1, keepdims=True)
    ones_col = jnp.ones((_B, 1), jnp.float32)
    a_aug = jnp.concatenate([-2.0 * e, n_vec, ones_col], axis=1)
    b_aug = jnp.concatenate([e, ones_col, n_vec], axis=1)

    acc_off = jnp.zeros((_T, _T), jnp.float32)
    acc_diag = jnp.zeros((_T, _T), jnp.float32)
    for bi in range(_NT):
        a_blk = a_aug[bi * _T:(bi + 1) * _T, :]
        idc = idx_f[bi * _T:(bi + 1) * _T, :]
        for bj in range(bi, _NT):
            b_blk = b_aug[bj * _T:(bj + 1) * _T, :]
            idr = idx_row[:, bj * _T:(bj + 1) * _T]
            d2 = jnp.maximum(
                jax.lax.dot_general(a_blk, b_blk, _CONTRACT_LAST,
                                    preferred_element_type=jnp.float32), 0.0)
            s = d2 * jax.lax.rsqrt(d2 + 1e-12)
            t = jnp.maximum(_MARGIN - s, 0.0)
            loss_t = jnp.where(idc == idr, d2, t * t)
            if bi == bj:
                acc_diag = acc_diag + loss_t
            else:
                acc_off = acc_off + loss_t
    tot = acc_off + 0.5 * acc_diag
    row_sums = jnp.sum(tot, axis=1, keepdims=True)
    total = jnp.sum(row_sums, axis=0, keepdims=True)
    out_ref[:, :] = total / _NPAIRS


def kernel(embeddings, labels):
    out = pl.pallas_call(
        _loss_kernel,
        out_shape=jax.ShapeDtypeStruct((1, 1), jnp.float32),
    )(embeddings, labels)
    return out[0, 0]


# speedup vs baseline: 1613.1133x; 1.0174x over previous
import jax
import jax.numpy as jnp
from jax.experimental import pallas as pl

_MARGIN = 1.0
_B = 1024
_T = 128
_NT = _B // _T
_NPAIRS = _B * (_B - 1) // 2
_CONTRACT_LAST = (((1,), (1,)), ((), ()))


def _loss_kernel(emb_ref, lab_ref, out_ref):
    e = emb_ref[:]
    lab = lab_ref[:]

    m = jnp.max(lab, axis=1, keepdims=True)
    col = jax.lax.broadcasted_iota(jnp.int32, (1, lab.shape[1]), 1)
    w = jax.lax.bitcast_convert_type((127 - col) << 23, jnp.float32)
    mw = jnp.where(lab == m, w, 0.0)
    z = jax.lax.dot_general(mw, jnp.ones((1, lab.shape[1]), jnp.float32),
                            _CONTRACT_LAST,
                            preferred_element_type=jnp.float32)
    zbits = jax.lax.bitcast_convert_type(z, jnp.int32)
    idx_f = (127 - (zbits >> 23)).astype(jnp.float32)

    one = jnp.ones((1, 1), jnp.float32)
    idx_row = jax.lax.dot_general(one, idx_f, _CONTRACT_LAST,
                                  preferred_element_type=jnp.float32)

    n_vec = jnp.sum(e * e, axis=1, keepdims=True)
    ones_col = jnp.ones((_B, 1), jnp.float32)
    a_aug = jnp.concatenate([-2.0 * e, n_vec, ones_col], axis=1)
    b_aug = jnp.concatenate([e, ones_col, n_vec], axis=1)

    acc_off = jnp.zeros((_T, _T), jnp.float32)
    acc_diag = jnp.zeros((_T, _T), jnp.float32)
    for bi in range(_NT):
        a_blk = a_aug[bi * _T:(bi + 1) * _T, :]
        idc = idx_f[bi * _T:(bi + 1) * _T, :]
        for bj in range(bi, _NT):
            b_blk = b_aug[bj * _T:(bj + 1) * _T, :]
            idr = idx_row[:, bj * _T:(bj + 1) * _T]
            d2 = jnp.maximum(
                jax.lax.dot_general(a_blk, b_blk, _CONTRACT_LAST,
                                    preferred_element_type=jnp.float32), 1e-12)
            s = d2 * jax.lax.rsqrt(d2)
            t = jnp.maximum(_MARGIN - s, 0.0)
            loss_t = jnp.where(idc == idr, d2, t * t)
            if bi == bj:
                acc_diag = acc_diag + loss_t
            else:
                acc_off = acc_off + loss_t
    tot = acc_off + 0.5 * acc_diag
    row_sums = jnp.sum(tot, axis=1, keepdims=True)
    total = jnp.sum(row_sums, axis=0, keepdims=True)
    out_ref[:, :] = total / _NPAIRS


def kernel(embeddings, labels):
    out = pl.pallas_call(
        _loss_kernel,
        out_shape=jax.ShapeDtypeStruct((1, 1), jnp.float32),
    )(embeddings, labels)
    return out[0, 0]
